# baseline (speedup 1.0000x reference)
import functools

import ml_dtypes
import numpy as np

import jax
import jax.numpy as jnp
from jax import lax
from jax.experimental import pallas as pl
from jax.experimental.pallas import tpu as pltpu
from jax.experimental.pallas import tpu_sc as plsc

GRID_XYZ = (100, 100, 8)
CDIM = 128
NEG = -1e30
_NEGB = int(np.asarray(NEG, dtype=ml_dtypes.bfloat16).view(np.uint16))
NEGPAIR = int(np.array((_NEGB << 16) | _NEGB, dtype=np.uint32).view(np.int32))

NPTS = 65536
PBLK = 2048
NBLK = NPTS // PBLK
RXY = 20480
NXY = 4 * RXY
NYZ = 6400
NXZ = 6400


def _accum_moments(u, i, s_ref, m_ref, acc_s, acc_m):
    @pl.when(i == 0)
    def _init():
        acc_s[...] = jnp.zeros_like(acc_s)
        acc_m[...] = jnp.zeros_like(acc_m)

    acc_s[...] += jnp.sum(u, axis=0, keepdims=True)
    acc_m[...] += lax.dot_general(u, u, (((0,), (0,)), ((), ())),
                                  preferred_element_type=jnp.float32)

    @pl.when(i == pl.num_programs(0) - 1)
    def _emit():
        s_ref[...] = acc_s[...]
        m_ref[...] = acc_m[...]


def _mm(a, w):
    return lax.dot_general(a, w, (((1,), (0,)), ((), ())),
                           preferred_element_type=jnp.float32)


def _moments_body(x_ref, s_ref, m_ref, acc_s, acc_m):
    _accum_moments(x_ref[...], pl.program_id(0), s_ref, m_ref, acc_s, acc_m)


def _mlp1_moments_body(x_ref, w1_ref, b1_ref, s_ref, m_ref, acc_s, acc_m):
    u = jax.nn.relu(_mm(x_ref[...], w1_ref[...]) + b1_ref[...])
    _accum_moments(u, pl.program_id(0), s_ref, m_ref, acc_s, acc_m)


def _mlp2_moments_body(x_ref, w1_ref, b1_ref, w2_ref, b2_ref,
                       s_ref, m_ref, acc_s, acc_m):
    u = jax.nn.relu(_mm(x_ref[...], w1_ref[...]) + b1_ref[...])
    u = jax.nn.relu(_mm(u, w2_ref[...]) + b2_ref[...])
    _accum_moments(u, pl.program_id(0), s_ref, m_ref, acc_s, acc_m)


def _moments(x, blk):
    n, d = x.shape
    return pl.pallas_call(
        _moments_body,
        grid=(n // blk,),
        in_specs=[pl.BlockSpec((blk, d), lambda i: (i, 0))],
        out_specs=[pl.BlockSpec((1, d), lambda i: (0, 0)),
                   pl.BlockSpec((d, d), lambda i: (0, 0))],
        out_shape=[jax.ShapeDtypeStruct((1, d), jnp.float32),
                   jax.ShapeDtypeStruct((d, d), jnp.float32)],
        scratch_shapes=[pltpu.VMEM((1, d), jnp.float32),
                        pltpu.VMEM((d, d), jnp.float32)],
    )(x)


def _mlp_moments(x, w1, b1, w2, b2, blk):
    n = x.shape[0]
    d = w1.shape[1] if w2 is None else w2.shape[1]
    args = [x, w1, b1] + ([] if w2 is None else [w2, b2])
    body = _mlp1_moments_body if w2 is None else _mlp2_moments_body
    return pl.pallas_call(
        body,
        grid=(n // blk,),
        in_specs=[pl.BlockSpec((blk, x.shape[1]), lambda i: (i, 0))]
        + [pl.BlockSpec(a.shape, lambda i: (0, 0)) for a in args[1:]],
        out_specs=[pl.BlockSpec((1, d), lambda i: (0, 0)),
                   pl.BlockSpec((d, d), lambda i: (0, 0))],
        out_shape=[jax.ShapeDtypeStruct((1, d), jnp.float32),
                   jax.ShapeDtypeStruct((d, d), jnp.float32)],
        scratch_shapes=[pltpu.VMEM((1, d), jnp.float32),
                        pltpu.VMEM((d, d), jnp.float32)],
    )(*args)


def _proc_body(x_ref, cam_ref, idx_ref, w1_ref, b1_ref, w2_ref, b2_ref,
               w3_ref, b3_ref, w4_ref, b4_ref, wc_ref, bc_ref,
               out_ref, key_ref):
    u = jax.nn.relu(_mm(x_ref[...], w1_ref[...]) + b1_ref[...])
    u = jax.nn.relu(_mm(u, w2_ref[...]) + b2_ref[...])
    u = jax.nn.relu(_mm(u, w3_ref[...]) + b3_ref[...])
    proc = (_mm(u, w4_ref[...]) + b4_ref[...]
            + _mm(cam_ref[...], wc_ref[...]) + bc_ref[...])
    out_ref[...] = proc.T

    gx = idx_ref[0:1, :]
    gy = idx_ref[1:2, :]
    gz = idx_ref[2:3, :]
    gb = idx_ref[3:4, :]
    xq = (gx * 41) >> 10
    yq = (gy * 41) >> 10
    zq = gz >> 1
    r_xy = (gb * 100 + gx) * 100 + gy
    r_yz = (gb * 100 + gy) * 8 + gz
    r_xz = (gb * 100 + gx) * 8 + gz
    kxy = zq * RXY + r_xy
    kyz = xq * 1600 + r_yz
    kxz = yq * 1600 + r_xz
    zero = jnp.zeros_like(kxy)
    key_ref[...] = jnp.concatenate(
        [kxy, kyz, kxz, zero, zero, zero, zero, zero], axis=0)


def _proc_pallas(x, cam, idxT, weights, blk):
    n = x.shape[0]
    return pl.pallas_call(
        _proc_body,
        grid=(n // blk,),
        in_specs=[pl.BlockSpec((blk, x.shape[1]), lambda i: (i, 0)),
                  pl.BlockSpec((blk, cam.shape[1]), lambda i: (i, 0)),
                  pl.BlockSpec((8, blk), lambda i: (0, i))]
        + [pl.BlockSpec(w.shape, lambda i: (0, 0)) for w in weights],
        out_specs=[pl.BlockSpec((CDIM, blk), lambda i: (0, i)),
                   pl.BlockSpec((8, blk), lambda i: (0, i))],
        out_shape=[jax.ShapeDtypeStruct((CDIM, n), jnp.float32),
                   jax.ShapeDtypeStruct((8, n), jnp.int32)],
    )(x, cam, idxT, *weights)


def _halves(w_i32):
    lo = lax.bitcast_convert_type(w_i32 << 16, jnp.float32)
    hi = lax.bitcast_convert_type(w_i32 & jnp.int32(-65536), jnp.float32)
    return lo, hi


def _pair_pend(w_i32, vlo, vhi):
    lo, hi = _halves(w_i32)
    return (lo < vlo) | (hi < vhi)


def _sc_rmw_retry(acc, k, v_i32):
    v_bf = plsc.bitcast(v_i32, jnp.bfloat16)
    vlo, vhi = _halves(v_i32)

    def cond(pend):
        return jnp.max(pend) > 0

    def body(pend):
        mask = pend > 0
        chk = plsc.load_gather(acc, [k])
        new = plsc.bitcast(
            jnp.maximum(plsc.bitcast(chk, jnp.bfloat16), v_bf), jnp.int32)
        plsc.store_scatter(acc, [k], new, mask=mask)
        chk2 = plsc.load_gather(acc, [k])
        return jnp.where(_pair_pend(chk2, vlo, vhi),
                         jnp.int32(1), jnp.int32(0))

    chk = plsc.load_gather(acc, [k])
    pend0 = jnp.where(_pair_pend(chk, vlo, vhi), jnp.int32(1), jnp.int32(0))
    lax.while_loop(cond, body, pend0)


def _make_sc_scatter():
    mesh = plsc.VectorSubcoreMesh(core_axis_name="c", subcore_axis_name="s")

    @functools.partial(
        pl.kernel,
        mesh=mesh,
        compiler_params=pltpu.CompilerParams(needs_layout_passes=False),
        out_type=[jax.ShapeDtypeStruct((CDIM // 2 * NXY,), jnp.int32),
                  jax.ShapeDtypeStruct((CDIM // 2 * NYZ,), jnp.int32),
                  jax.ShapeDtypeStruct((CDIM // 2 * NXZ,), jnp.int32)],
        scratch_types=[pltpu.VMEM((NXY,), jnp.int32),
                       pltpu.VMEM((NYZ,), jnp.int32),
                       pltpu.VMEM((NXZ,), jnp.int32)]
        + [pltpu.VMEM((PBLK,), jnp.int32) for _ in range(8)]
        + [pltpu.SemaphoreType.DMA, pltpu.SemaphoreType.DMA],
    )
    def sck(kxy_hbm, kyz_hbm, kxz_hbm, vt_hbm, oxy_hbm, oyz_hbm, oxz_hbm,
            acc_xy, acc_yz, acc_xz,
            kxy0, kyz0, kxz0, kxy1, kyz1, kxz1, vb0, vb1, sem0, sem1):
        wid = lax.axis_index("s") * 2 + lax.axis_index("c")
        kbufs = ((kxy0, kyz0, kxz0, vb0, sem0), (kxy1, kyz1, kxz1, vb1, sem1))

        def start_blk(b, ch, bufs):
            kx, ky, kz, vb, sem = bufs
            off = b * PBLK
            pltpu.async_copy(kxy_hbm.at[pl.ds(off, PBLK)], kx, sem)
            pltpu.async_copy(kyz_hbm.at[pl.ds(off, PBLK)], ky, sem)
            pltpu.async_copy(kxz_hbm.at[pl.ds(off, PBLK)], kz, sem)
            pltpu.async_copy(vt_hbm.at[pl.ds(ch * NPTS + off, PBLK)], vb, sem)

        def wait_blk(b, ch, bufs):
            kx, ky, kz, vb, sem = bufs
            off = b * PBLK
            pltpu.make_async_copy(kxy_hbm.at[pl.ds(off, PBLK)], kx, sem).wait()
            pltpu.make_async_copy(kyz_hbm.at[pl.ds(off, PBLK)], ky, sem).wait()
            pltpu.make_async_copy(kxz_hbm.at[pl.ds(off, PBLK)], kz, sem).wait()
            pltpu.make_async_copy(
                vt_hbm.at[pl.ds(ch * NPTS + off, PBLK)], vb, sem).wait()

        def compute_blk(bufs):
            kx, ky, kz, vb, _ = bufs

            def vreg_fast(j, bad):
                U = 8
                ss = [(j * U + t) * 16 for t in range(U)]
                vw = [vb[pl.ds(s, 16)] for s in ss]
                vs = [plsc.bitcast(w, jnp.bfloat16) for w in vw]
                vh = [_halves(w) for w in vw]
                planes = ((acc_xy, [kx[pl.ds(s, 16)] for s in ss]),
                          (acc_yz, [ky[pl.ds(s, 16)] for s in ss]),
                          (acc_xz, [kz[pl.ds(s, 16)] for s in ss]))

                def pmax(w_i32, v_bf):
                    c = plsc.bitcast(w_i32, jnp.bfloat16)
                    return plsc.bitcast(jnp.maximum(c, v_bf), jnp.int32)

                cur = [[plsc.load_gather(a, [ks[t]]) for t in range(U)]
                       for a, ks in planes]
                for p, (a, ks) in enumerate(planes):
                    for t in range(U):
                        plsc.store_scatter(a, [ks[t]], pmax(cur[p][t], vs[t]))
                for _ in range(2):
                    chk = [[plsc.load_gather(a, [ks[t]]) for t in range(U)]
                           for a, ks in planes]
                    pend = [[_pair_pend(chk[p][t], vh[t][0], vh[t][1])
                             for t in range(U)]
                            for p in range(3)]
                    for p, (a, ks) in enumerate(planes):
                        for t in range(U):
                            plsc.store_scatter(a, [ks[t]],
                                               pmax(chk[p][t], vs[t]),
                                               mask=pend[p][t])
                fin = [[plsc.load_gather(a, [ks[t]]) for t in range(U)]
                       for a, ks in planes]
                for p in range(3):
                    for t in range(U):
                        bad = bad | _pair_pend(fin[p][t], vh[t][0], vh[t][1])
                return bad

            bad = lax.fori_loop(0, PBLK // 128, vreg_fast,
                                jnp.zeros((16,), jnp.bool_))
            nbad = plsc.all_reduce_population_count(bad)

            @pl.when(jnp.max(nbad) > 0)
            def _slow_sweep():
                def vreg_slow(j, carry):
                    s = j * 16
                    v = vb[pl.ds(s, 16)]
                    _sc_rmw_retry(acc_xy, kx[pl.ds(s, 16)], v)
                    _sc_rmw_retry(acc_yz, ky[pl.ds(s, 16)], v)
                    _sc_rmw_retry(acc_xz, kz[pl.ds(s, 16)], v)
                    return carry

                lax.fori_loop(0, PBLK // 16, vreg_slow, 0)

        def round_step(r, carry):
            ch = r * 32 + wid

            def init_acc(acc, nslots):
                neg = jnp.full((16,), NEGPAIR, jnp.int32)

                def ini(i, c):
                    base = i * 128
                    for t in range(8):
                        acc[pl.ds(base + t * 16, 16)] = neg
                    return c
                lax.fori_loop(0, nslots // 128, ini, 0)

            init_acc(acc_xy, NXY)
            init_acc(acc_yz, NYZ)
            init_acc(acc_xz, NXZ)

            start_blk(0, ch, kbufs[0])

            def super_step(i, carry):
                b0 = i * 2

                @pl.when(b0 + 1 < NBLK)
                def _s1():
                    start_blk(b0 + 1, ch, kbufs[1])

                wait_blk(b0, ch, kbufs[0])
                compute_blk(kbufs[0])

                @pl.when(b0 + 2 < NBLK)
                def _s0():
                    start_blk(b0 + 2, ch, kbufs[0])

                wait_blk(b0 + 1, ch, kbufs[1])
                compute_blk(kbufs[1])
                return carry

            lax.fori_loop(0, NBLK // 2, super_step, 0)

            pltpu.sync_copy(acc_xy, oxy_hbm.at[pl.ds(ch * NXY, NXY)])
            pltpu.sync_copy(acc_yz, oyz_hbm.at[pl.ds(ch * NYZ, NYZ)])
            pltpu.sync_copy(acc_xz, oxz_hbm.at[pl.ds(ch * NXZ, NXZ)])
            return carry

        lax.fori_loop(0, CDIM // 2 // 32, round_step, 0)

    return sck


_sc_scatter = _make_sc_scatter()


def _plane_mlp_body(x_ref, w1_ref, b1_ref, w2_ref, b2_ref, out_ref):
    xi = x_ref[...].reshape(2 * CDIM, x_ref.shape[2])
    lo = lax.bitcast_convert_type(xi << 16, jnp.float32)
    hi = lax.bitcast_convert_type(xi & jnp.int32(-65536), jnp.float32)
    x = jnp.concatenate([lo, hi], axis=0)
    x = jnp.where(x < -1e29, 0.0, x)
    u = jax.nn.relu(_mm(w1_ref[...], x) + b1_ref[...])
    out_ref[...] = _mm(w2_ref[...], u) + b2_ref[...]


def _plane_mlp(xcm, w1p, b1c, w2t, b2c, blk):
    r = xcm.shape[2]
    return pl.pallas_call(
        _plane_mlp_body,
        grid=(r // blk,),
        in_specs=[pl.BlockSpec((CDIM // 2, 4, blk), lambda i: (0, 0, i)),
                  pl.BlockSpec(w1p.shape, lambda i: (0, 0)),
                  pl.BlockSpec(b1c.shape, lambda i: (0, 0)),
                  pl.BlockSpec(w2t.shape, lambda i: (0, 0)),
                  pl.BlockSpec(b2c.shape, lambda i: (0, 0))],
        out_specs=pl.BlockSpec((CDIM, blk), lambda i: (0, i)),
        out_shape=jax.ShapeDtypeStruct((CDIM, r), jnp.float32),
    )(xcm, w1p, b1c, w2t, b2c)


def _perm_w1(w1):
    w = w1.reshape(4, CDIM, CDIM)
    lo = jnp.transpose(w[:, 0::2, :], (1, 0, 2)).reshape(2 * CDIM, CDIM)
    hi = jnp.transpose(w[:, 1::2, :], (1, 0, 2)).reshape(2 * CDIM, CDIM)
    return jnp.concatenate([lo, hi], axis=0).T


def _fold_bn_next(mu, cov, w, b, p_next):
    mean_h = mu @ w + b
    var_h = jnp.sum(w * (cov @ w), axis=0)
    a = p_next["g"] / jnp.sqrt(var_h + 1e-5)
    c = p_next["b"] - mean_h * a
    return a, c


def kernel(points, grid_ind, cam_point_features, params):
    X, Y, Z = GRID_XYZ
    B, N = points.shape[0], points.shape[1]
    M = B * N
    fea = points[..., :5].reshape(M, 5)
    feap = jnp.pad(fea, ((0, 0), (0, 3)))
    camf = cam_point_features.reshape(M, cam_point_features.shape[-1])
    ind = grid_ind.reshape(M, 3).astype(jnp.int32)
    bidx = jnp.repeat(jnp.arange(B, dtype=jnp.int32), N)
    idxT = jnp.concatenate(
        [ind.T, bidx[None, :], jnp.zeros((4, M), jnp.int32)], axis=0)

    s0, m0 = _moments(feap, 8192)
    mu0 = s0[0] / M
    cov0 = m0 / M - jnp.outer(mu0, mu0)
    g0 = jnp.pad(params["bn0"]["g"], (0, 3))
    bb0 = jnp.pad(params["bn0"]["b"], (0, 3))
    a0 = jnp.where(jnp.arange(8) < 5,
                   g0 / jnp.sqrt(jnp.diagonal(cov0) + 1e-5), 0.0)
    c0 = bb0 - mu0 * a0

    w1 = jnp.pad(params["fc1"]["w"], ((0, 3), (0, 0)))
    w1f = a0[:, None] * w1
    b1f = c0 @ w1 + params["fc1"]["b"]
    mean1 = mu0 @ w1f + b1f
    var1 = jnp.sum(w1f * (cov0 @ w1f), axis=0)
    a1 = params["bn1"]["g"] / jnp.sqrt(var1 + 1e-5)
    c1 = params["bn1"]["b"] - mean1 * a1
    w1ff = w1f * a1[None, :]
    b1ff = b1f * a1 + c1

    s1, m1 = _mlp_moments(feap, w1ff, b1ff.reshape(1, -1), None, None, 8192)
    mu1 = s1[0] / M
    cov1 = m1 / M - jnp.outer(mu1, mu1)
    a2, c2 = _fold_bn_next(mu1, cov1, params["fc2"]["w"], params["fc2"]["b"],
                           params["bn2"])
    w2ff = params["fc2"]["w"] * a2[None, :]
    b2ff = params["fc2"]["b"] * a2 + c2

    s2, m2 = _mlp_moments(feap, w1ff, b1ff.reshape(1, -1),
                          w2ff, b2ff.reshape(1, -1), 8192)
    mu2 = s2[0] / M
    cov2 = m2 / M - jnp.outer(mu2, mu2)
    a3, c3 = _fold_bn_next(mu2, cov2, params["fc3"]["w"], params["fc3"]["b"],
                           params["bn3"])
    w3ff = params["fc3"]["w"] * a3[None, :]
    b3ff = params["fc3"]["b"] * a3 + c3

    weights = [w1ff, b1ff.reshape(1, -1), w2ff, b2ff.reshape(1, -1),
               w3ff, b3ff.reshape(1, -1),
               params["fc4"]["w"], params["fc4"]["b"].reshape(1, CDIM),
               params["cam"]["w"], params["cam"]["b"].reshape(1, CDIM)]
    proct, keys = _proc_pallas(feap, camf, idxT, weights, 2048)

    u16 = lax.bitcast_convert_type(proct.astype(jnp.bfloat16), jnp.uint16)
    lo = u16[0::2].astype(jnp.uint32)
    hi = u16[1::2].astype(jnp.uint32)
    vt = lax.bitcast_convert_type((hi << 16) | lo, jnp.int32).reshape(
        CDIM // 2 * M)

    oxy, oyz, oxz = _sc_scatter(keys[0], keys[1], keys[2], vt)

    txy = _plane_mlp(oxy.reshape(CDIM // 2, 4, RXY),
                     _perm_w1(params["xy1"]["w"]),
                     params["xy1"]["b"].reshape(CDIM, 1),
                     params["xy2"]["w"].T,
                     params["xy2"]["b"].reshape(CDIM, 1), 2048)
    txy = txy[:, :B * X * Y]
    tyz = _plane_mlp(oyz.reshape(CDIM // 2, 4, B * Y * Z),
                     _perm_w1(params["yz1"]["w"]),
                     params["yz1"]["b"].reshape(CDIM, 1),
                     params["yz2"]["w"].T,
                     params["yz2"]["b"].reshape(CDIM, 1), 1600)
    txz = _plane_mlp(oxz.reshape(CDIM // 2, 4, B * X * Z),
                     _perm_w1(params["xz1"]["w"]),
                     params["xz1"]["b"].reshape(CDIM, 1),
                     params["xz2"]["w"].T,
                     params["xz2"]["b"].reshape(CDIM, 1), 1600)

    txy = jnp.swapaxes(txy.reshape(CDIM, B, X, Y), 0, 1)
    tyz = jnp.swapaxes(tyz.reshape(CDIM, B, Y, Z), 0, 1)
    txz = jnp.swapaxes(txz.reshape(CDIM, B, X, Z), 0, 1)
    return (txy, tyz, txz)


# speedup vs baseline: 1.1167x; 1.1167x over previous
import functools

import ml_dtypes
import numpy as np

import jax
import jax.numpy as jnp
from jax import lax
from jax.experimental import pallas as pl
from jax.experimental.pallas import tpu as pltpu
from jax.experimental.pallas import tpu_sc as plsc

GRID_XYZ = (100, 100, 8)
CDIM = 128
NEG = -1e30
_NEGB = int(np.asarray(NEG, dtype=ml_dtypes.bfloat16).view(np.uint16))
NEGPAIR = int(np.array((_NEGB << 16) | _NEGB, dtype=np.uint32).view(np.int32))

NPTS = 65536
PBLK = 2048
NBLK = NPTS // PBLK
RXY = 20480
NXY = 4 * RXY
NYZ = 6400
NXZ = 6400


def _accum_moments(u, i, s_ref, m_ref, acc_s, acc_m):
    @pl.when(i == 0)
    def _init():
        acc_s[...] = jnp.zeros_like(acc_s)
        acc_m[...] = jnp.zeros_like(acc_m)

    acc_s[...] += jnp.sum(u, axis=0, keepdims=True)
    acc_m[...] += lax.dot_general(u, u, (((0,), (0,)), ((), ())),
                                  preferred_element_type=jnp.float32)

    @pl.when(i == pl.num_programs(0) - 1)
    def _emit():
        s_ref[...] = acc_s[...]
        m_ref[...] = acc_m[...]


def _mm(a, w):
    return lax.dot_general(a, w, (((1,), (0,)), ((), ())),
                           preferred_element_type=jnp.float32)


def _moments_body(x_ref, s_ref, m_ref, acc_s, acc_m):
    _accum_moments(x_ref[...], pl.program_id(0), s_ref, m_ref, acc_s, acc_m)


def _mlp1_moments_body(x_ref, w1_ref, b1_ref, s_ref, m_ref, acc_s, acc_m):
    u = jax.nn.relu(_mm(x_ref[...], w1_ref[...]) + b1_ref[...])
    _accum_moments(u, pl.program_id(0), s_ref, m_ref, acc_s, acc_m)


def _mlp2_moments_body(x_ref, w1_ref, b1_ref, w2_ref, b2_ref,
                       s_ref, m_ref, acc_s, acc_m):
    u = jax.nn.relu(_mm(x_ref[...], w1_ref[...]) + b1_ref[...])
    u = jax.nn.relu(_mm(u, w2_ref[...]) + b2_ref[...])
    _accum_moments(u, pl.program_id(0), s_ref, m_ref, acc_s, acc_m)


def _moments(x, blk):
    n, d = x.shape
    return pl.pallas_call(
        _moments_body,
        grid=(n // blk,),
        in_specs=[pl.BlockSpec((blk, d), lambda i: (i, 0))],
        out_specs=[pl.BlockSpec((1, d), lambda i: (0, 0)),
                   pl.BlockSpec((d, d), lambda i: (0, 0))],
        out_shape=[jax.ShapeDtypeStruct((1, d), jnp.float32),
                   jax.ShapeDtypeStruct((d, d), jnp.float32)],
        scratch_shapes=[pltpu.VMEM((1, d), jnp.float32),
                        pltpu.VMEM((d, d), jnp.float32)],
    )(x)


def _mlp_moments(x, w1, b1, w2, b2, blk):
    n = x.shape[0]
    d = w1.shape[1] if w2 is None else w2.shape[1]
    args = [x, w1, b1] + ([] if w2 is None else [w2, b2])
    body = _mlp1_moments_body if w2 is None else _mlp2_moments_body
    return pl.pallas_call(
        body,
        grid=(n // blk,),
        in_specs=[pl.BlockSpec((blk, x.shape[1]), lambda i: (i, 0))]
        + [pl.BlockSpec(a.shape, lambda i: (0, 0)) for a in args[1:]],
        out_specs=[pl.BlockSpec((1, d), lambda i: (0, 0)),
                   pl.BlockSpec((d, d), lambda i: (0, 0))],
        out_shape=[jax.ShapeDtypeStruct((1, d), jnp.float32),
                   jax.ShapeDtypeStruct((d, d), jnp.float32)],
        scratch_shapes=[pltpu.VMEM((1, d), jnp.float32),
                        pltpu.VMEM((d, d), jnp.float32)],
    )(*args)


def _proc_body(x_ref, cam_ref, idx_ref, w1_ref, b1_ref, w2_ref, b2_ref,
               w3_ref, b3_ref, w4_ref, b4_ref, wc_ref, bc_ref,
               out_ref, key_ref):
    u = jax.nn.relu(_mm(x_ref[...], w1_ref[...]) + b1_ref[...])
    u = jax.nn.relu(_mm(u, w2_ref[...]) + b2_ref[...])
    u = jax.nn.relu(_mm(u, w3_ref[...]) + b3_ref[...])
    proc = (_mm(u, w4_ref[...]) + b4_ref[...]
            + _mm(cam_ref[...], wc_ref[...]) + bc_ref[...])
    out_ref[...] = proc.T

    gx = idx_ref[0:1, :]
    gy = idx_ref[1:2, :]
    gz = idx_ref[2:3, :]
    gb = idx_ref[3:4, :]
    xq = (gx * 41) >> 10
    yq = (gy * 41) >> 10
    zq = gz >> 1
    r_xy = (gb * 100 + gx) * 100 + gy
    r_yz = (gb * 100 + gy) * 8 + gz
    r_xz = (gb * 100 + gx) * 8 + gz
    kxy = zq * RXY + r_xy
    kyz = xq * 1600 + r_yz
    kxz = yq * 1600 + r_xz
    zero = jnp.zeros_like(kxy)
    key_ref[...] = jnp.concatenate(
        [kxy, kyz, kxz, zero, zero, zero, zero, zero], axis=0)


def _proc_pallas(x, cam, idxT, weights, blk):
    n = x.shape[0]
    return pl.pallas_call(
        _proc_body,
        grid=(n // blk,),
        in_specs=[pl.BlockSpec((blk, x.shape[1]), lambda i: (i, 0)),
                  pl.BlockSpec((blk, cam.shape[1]), lambda i: (i, 0)),
                  pl.BlockSpec((8, blk), lambda i: (0, i))]
        + [pl.BlockSpec(w.shape, lambda i: (0, 0)) for w in weights],
        out_specs=[pl.BlockSpec((CDIM, blk), lambda i: (0, i)),
                   pl.BlockSpec((8, blk), lambda i: (0, i))],
        out_shape=[jax.ShapeDtypeStruct((CDIM, n), jnp.float32),
                   jax.ShapeDtypeStruct((8, n), jnp.int32)],
    )(x, cam, idxT, *weights)


def _sc_rmw_retry(acc, k, v):
    def cond(pend):
        return jnp.max(pend) > 0

    def body(pend):
        mask = pend > 0
        plsc.store_scatter(acc, [k], v, mask=mask)
        chk = plsc.load_gather(acc, [k])
        return jnp.where(chk < v, jnp.int32(1), jnp.int32(0))

    chk = plsc.load_gather(acc, [k])
    pend0 = jnp.where(chk < v, jnp.int32(1), jnp.int32(0))
    lax.while_loop(cond, body, pend0)


def _make_sc_scatter():
    mesh = plsc.VectorSubcoreMesh(core_axis_name="c", subcore_axis_name="s")

    @functools.partial(
        pl.kernel,
        mesh=mesh,
        compiler_params=pltpu.CompilerParams(needs_layout_passes=False),
        out_type=[jax.ShapeDtypeStruct((CDIM * NXY,), jnp.float32),
                  jax.ShapeDtypeStruct((CDIM * NYZ,), jnp.float32),
                  jax.ShapeDtypeStruct((CDIM * NXZ,), jnp.float32)],
        scratch_types=[pltpu.VMEM((NXY,), jnp.float32),
                       pltpu.VMEM((NYZ,), jnp.float32),
                       pltpu.VMEM((NXZ,), jnp.float32)]
        + [pltpu.VMEM((PBLK,), jnp.int32) for _ in range(6)]
        + [pltpu.VMEM((PBLK,), jnp.float32) for _ in range(2)]
        + [pltpu.SemaphoreType.DMA, pltpu.SemaphoreType.DMA],
    )
    def sck(kxy_hbm, kyz_hbm, kxz_hbm, vt_hbm, oxy_hbm, oyz_hbm, oxz_hbm,
            acc_xy, acc_yz, acc_xz,
            kxy0, kyz0, kxz0, kxy1, kyz1, kxz1, vb0, vb1, sem0, sem1):
        wid = lax.axis_index("s") * 2 + lax.axis_index("c")
        kbufs = ((kxy0, kyz0, kxz0, vb0, sem0), (kxy1, kyz1, kxz1, vb1, sem1))

        def start_blk(b, ch, bufs):
            kx, ky, kz, vb, sem = bufs
            off = b * PBLK
            pltpu.async_copy(kxy_hbm.at[pl.ds(off, PBLK)], kx, sem)
            pltpu.async_copy(kyz_hbm.at[pl.ds(off, PBLK)], ky, sem)
            pltpu.async_copy(kxz_hbm.at[pl.ds(off, PBLK)], kz, sem)
            pltpu.async_copy(vt_hbm.at[pl.ds(ch * NPTS + off, PBLK)], vb, sem)

        def wait_blk(b, ch, bufs):
            kx, ky, kz, vb, sem = bufs
            off = b * PBLK
            pltpu.make_async_copy(kxy_hbm.at[pl.ds(off, PBLK)], kx, sem).wait()
            pltpu.make_async_copy(kyz_hbm.at[pl.ds(off, PBLK)], ky, sem).wait()
            pltpu.make_async_copy(kxz_hbm.at[pl.ds(off, PBLK)], kz, sem).wait()
            pltpu.make_async_copy(
                vt_hbm.at[pl.ds(ch * NPTS + off, PBLK)], vb, sem).wait()

        def compute_blk(bufs):
            kx, ky, kz, vb, _ = bufs

            def vreg_fast(j, bad):
                U = 8
                ss = [(j * U + t) * 16 for t in range(U)]
                vs = [vb[pl.ds(s, 16)] for s in ss]
                planes = ((acc_xy, [kx[pl.ds(s, 16)] for s in ss]),
                          (acc_yz, [ky[pl.ds(s, 16)] for s in ss]),
                          (acc_xz, [kz[pl.ds(s, 16)] for s in ss]))
                cur = [[plsc.load_gather(a, [ks[t]]) for t in range(U)]
                       for a, ks in planes]
                for p, (a, ks) in enumerate(planes):
                    for t in range(U):
                        plsc.store_scatter(a, [ks[t]],
                                           jnp.maximum(cur[p][t], vs[t]))
                for _ in range(2):
                    chk = [[plsc.load_gather(a, [ks[t]]) for t in range(U)]
                           for a, ks in planes]
                    pend = [[chk[p][t] < vs[t] for t in range(U)]
                            for p in range(3)]
                    for p, (a, ks) in enumerate(planes):
                        for t in range(U):
                            plsc.store_scatter(a, [ks[t]], vs[t],
                                               mask=pend[p][t])
                fin = [[plsc.load_gather(a, [ks[t]]) for t in range(U)]
                       for a, ks in planes]
                for p in range(3):
                    for t in range(U):
                        bad = bad | (fin[p][t] < vs[t])
                return bad

            bad = lax.fori_loop(0, PBLK // 128, vreg_fast,
                                jnp.zeros((16,), jnp.bool_))
            nbad = plsc.all_reduce_population_count(bad)

            @pl.when(jnp.max(nbad) > 0)
            def _slow_sweep():
                def vreg_slow(j, carry):
                    s = j * 16
                    v = vb[pl.ds(s, 16)]
                    _sc_rmw_retry(acc_xy, kx[pl.ds(s, 16)], v)
                    _sc_rmw_retry(acc_yz, ky[pl.ds(s, 16)], v)
                    _sc_rmw_retry(acc_xz, kz[pl.ds(s, 16)], v)
                    return carry

                lax.fori_loop(0, PBLK // 16, vreg_slow, 0)

        def round_step(r, carry):
            ch = r * 32 + wid

            def init_acc(acc, nslots):
                neg = jnp.full((16,), NEG, jnp.float32)

                def ini(i, c):
                    base = i * 128
                    for t in range(8):
                        acc[pl.ds(base + t * 16, 16)] = neg
                    return c
                lax.fori_loop(0, nslots // 128, ini, 0)

            init_acc(acc_xy, NXY)
            init_acc(acc_yz, NYZ)
            init_acc(acc_xz, NXZ)

            start_blk(0, ch, kbufs[0])

            def super_step(i, carry):
                b0 = i * 2

                @pl.when(b0 + 1 < NBLK)
                def _s1():
                    start_blk(b0 + 1, ch, kbufs[1])

                wait_blk(b0, ch, kbufs[0])
                compute_blk(kbufs[0])

                @pl.when(b0 + 2 < NBLK)
                def _s0():
                    start_blk(b0 + 2, ch, kbufs[0])

                wait_blk(b0 + 1, ch, kbufs[1])
                compute_blk(kbufs[1])
                return carry

            lax.fori_loop(0, NBLK // 2, super_step, 0)

            pltpu.sync_copy(acc_xy, oxy_hbm.at[pl.ds(ch * NXY, NXY)])
            pltpu.sync_copy(acc_yz, oyz_hbm.at[pl.ds(ch * NYZ, NYZ)])
            pltpu.sync_copy(acc_xz, oxz_hbm.at[pl.ds(ch * NXZ, NXZ)])
            return carry

        lax.fori_loop(0, CDIM // 32, round_step, 0)

    return sck


_sc_scatter = _make_sc_scatter()


def _plane_mlp_body(x_ref, w1_ref, b1_ref, w2_ref, b2_ref, out_ref):
    x = x_ref[...].reshape(4 * CDIM, x_ref.shape[2])
    x = jnp.where(x < -1e29, 0.0, x)
    u = jax.nn.relu(_mm(w1_ref[...], x) + b1_ref[...])
    out_ref[...] = _mm(w2_ref[...], u) + b2_ref[...]


def _plane_mlp(xcm, w1p, b1c, w2t, b2c, blk):
    r = xcm.shape[2]
    return pl.pallas_call(
        _plane_mlp_body,
        grid=(r // blk,),
        in_specs=[pl.BlockSpec((CDIM, 4, blk), lambda i: (0, 0, i)),
                  pl.BlockSpec(w1p.shape, lambda i: (0, 0)),
                  pl.BlockSpec(b1c.shape, lambda i: (0, 0)),
                  pl.BlockSpec(w2t.shape, lambda i: (0, 0)),
                  pl.BlockSpec(b2c.shape, lambda i: (0, 0))],
        out_specs=pl.BlockSpec((CDIM, blk), lambda i: (0, i)),
        out_shape=jax.ShapeDtypeStruct((CDIM, r), jnp.float32),
    )(xcm, w1p, b1c, w2t, b2c)


def _perm_w1(w1):
    return jnp.transpose(w1.reshape(4, CDIM, CDIM), (2, 1, 0)).reshape(CDIM, 4 * CDIM)


def _fold_bn_next(mu, cov, w, b, p_next):
    mean_h = mu @ w + b
    var_h = jnp.sum(w * (cov @ w), axis=0)
    a = p_next["g"] / jnp.sqrt(var_h + 1e-5)
    c = p_next["b"] - mean_h * a
    return a, c


def kernel(points, grid_ind, cam_point_features, params):
    X, Y, Z = GRID_XYZ
    B, N = points.shape[0], points.shape[1]
    M = B * N
    fea = points[..., :5].reshape(M, 5)
    feap = jnp.pad(fea, ((0, 0), (0, 3)))
    camf = cam_point_features.reshape(M, cam_point_features.shape[-1])
    ind = grid_ind.reshape(M, 3).astype(jnp.int32)
    bidx = jnp.repeat(jnp.arange(B, dtype=jnp.int32), N)
    idxT = jnp.concatenate(
        [ind.T, bidx[None, :], jnp.zeros((4, M), jnp.int32)], axis=0)

    s0, m0 = _moments(feap, 8192)
    mu0 = s0[0] / M
    cov0 = m0 / M - jnp.outer(mu0, mu0)
    g0 = jnp.pad(params["bn0"]["g"], (0, 3))
    bb0 = jnp.pad(params["bn0"]["b"], (0, 3))
    a0 = jnp.where(jnp.arange(8) < 5,
                   g0 / jnp.sqrt(jnp.diagonal(cov0) + 1e-5), 0.0)
    c0 = bb0 - mu0 * a0

    w1 = jnp.pad(params["fc1"]["w"], ((0, 3), (0, 0)))
    w1f = a0[:, None] * w1
    b1f = c0 @ w1 + params["fc1"]["b"]
    mean1 = mu0 @ w1f + b1f
    var1 = jnp.sum(w1f * (cov0 @ w1f), axis=0)
    a1 = params["bn1"]["g"] / jnp.sqrt(var1 + 1e-5)
    c1 = params["bn1"]["b"] - mean1 * a1
    w1ff = w1f * a1[None, :]
    b1ff = b1f * a1 + c1

    s1, m1 = _mlp_moments(feap, w1ff, b1ff.reshape(1, -1), None, None, 8192)
    mu1 = s1[0] / M
    cov1 = m1 / M - jnp.outer(mu1, mu1)
    a2, c2 = _fold_bn_next(mu1, cov1, params["fc2"]["w"], params["fc2"]["b"],
                           params["bn2"])
    w2ff = params["fc2"]["w"] * a2[None, :]
    b2ff = params["fc2"]["b"] * a2 + c2

    s2, m2 = _mlp_moments(feap, w1ff, b1ff.reshape(1, -1),
                          w2ff, b2ff.reshape(1, -1), 8192)
    mu2 = s2[0] / M
    cov2 = m2 / M - jnp.outer(mu2, mu2)
    a3, c3 = _fold_bn_next(mu2, cov2, params["fc3"]["w"], params["fc3"]["b"],
                           params["bn3"])
    w3ff = params["fc3"]["w"] * a3[None, :]
    b3ff = params["fc3"]["b"] * a3 + c3

    weights = [w1ff, b1ff.reshape(1, -1), w2ff, b2ff.reshape(1, -1),
               w3ff, b3ff.reshape(1, -1),
               params["fc4"]["w"], params["fc4"]["b"].reshape(1, CDIM),
               params["cam"]["w"], params["cam"]["b"].reshape(1, CDIM)]
    proct, keys = _proc_pallas(feap, camf, idxT, weights, 2048)

    oxy, oyz, oxz = _sc_scatter(keys[0], keys[1], keys[2],
                                proct.reshape(CDIM * M))

    txy = _plane_mlp(oxy.reshape(CDIM, 4, RXY),
                     _perm_w1(params["xy1"]["w"]),
                     params["xy1"]["b"].reshape(CDIM, 1),
                     params["xy2"]["w"].T,
                     params["xy2"]["b"].reshape(CDIM, 1), 2048)
    txy = txy[:, :B * X * Y]
    tyz = _plane_mlp(oyz.reshape(CDIM, 4, B * Y * Z),
                     _perm_w1(params["yz1"]["w"]),
                     params["yz1"]["b"].reshape(CDIM, 1),
                     params["yz2"]["w"].T,
                     params["yz2"]["b"].reshape(CDIM, 1), 1600)
    txz = _plane_mlp(oxz.reshape(CDIM, 4, B * X * Z),
                     _perm_w1(params["xz1"]["w"]),
                     params["xz1"]["b"].reshape(CDIM, 1),
                     params["xz2"]["w"].T,
                     params["xz2"]["b"].reshape(CDIM, 1), 1600)

    txy = jnp.swapaxes(txy.reshape(CDIM, B, X, Y), 0, 1)
    tyz = jnp.swapaxes(tyz.reshape(CDIM, B, Y, Z), 0, 1)
    txz = jnp.swapaxes(txz.reshape(CDIM, B, X, Z), 0, 1)
    return (txy, tyz, txz)
